# Initial kernel scaffold; baseline (speedup 1.0000x reference)
import jax
import jax.numpy as jnp
from jax.experimental import pallas as pl


def kernel(z, edge_index):
    raise NotImplementedError("write your pallas kernel here")


# speedup vs baseline: 7.6902x; 7.6902x over previous
import jax
import jax.numpy as jnp
from jax import lax
from jax.experimental import pallas as pl
from jax.experimental.pallas import tpu as pltpu
from jax.experimental.pallas import tpu_sc as plsc

_NC = 2
_NS = 16
_NW = _NC * _NS
_L = 16

_E = 320000
_D = 128
_EW = _E // _NW
_C = 80
_G = _EW // _C


def _sc_body(z_hbm, src_hbm, dst_hbm, out_hbm,
             idx_s, idx_d, rows_s, rows_d, out_all, t_ref,
             sem_s0, sem_s1, sem_d0, sem_d1):
    wid = lax.axis_index("s") * _NC + lax.axis_index("c")
    base = wid * _EW

    row_iota = lax.iota(jnp.int32, _L)
    flat_iota = row_iota * _L

    pltpu.sync_copy(src_hbm.at[pl.ds(base, _EW)], idx_s)
    pltpu.sync_copy(dst_hbm.at[pl.ds(base, _EW)], idx_d)

    sems_s = (sem_s0, sem_s1)
    sems_d = (sem_d0, sem_d1)

    def start(g, b):
        cs = pltpu.async_copy(z_hbm.at[idx_s.at[pl.ds(g * _C, _C)]],
                              rows_s.at[b], sems_s[b])
        cd = pltpu.async_copy(z_hbm.at[idx_d.at[pl.ds(g * _C, _C)]],
                              rows_d.at[b], sems_d[b])
        return cs, cd

    def wait(b):
        pltpu.make_async_copy(z_hbm.at[idx_s.at[pl.ds(0, _C)]],
                              rows_s.at[b], sems_s[b]).wait()
        pltpu.make_async_copy(z_hbm.at[idx_d.at[pl.ds(0, _C)]],
                              rows_d.at[b], sems_d[b]).wait()

    def compute(g, b):
        rs = rows_s.at[b]
        rd = rows_d.at[b]

        def blk(k, carry):
            e0 = k * _L
            for j in range(_L):
                acc = rs[e0 + j, pl.ds(0, _L)] * rd[e0 + j, pl.ds(0, _L)]
                for q in range(1, _D // _L):
                    acc = acc + (rs[e0 + j, pl.ds(q * _L, _L)]
                                 * rd[e0 + j, pl.ds(q * _L, _L)])
                t_ref[pl.ds(j * _L, _L)] = acc
            res = plsc.load_gather(t_ref, [flat_iota])
            for d in range(1, _L):
                res = res + plsc.load_gather(t_ref, [flat_iota + d])
            e = jnp.exp(-jnp.abs(res))
            a = 1.0 / (1.0 + e)
            out_all[pl.ds(g * _C + e0, _L)] = jnp.where(res >= 0, a, 1.0 - a)
            return carry

        lax.fori_loop(0, _C // _L, blk, 0)

    start(0, 0)

    def step2(gg, carry):
        for b in range(2):
            g = gg + b
            wait(b)

            @pl.when(g + 1 < _G)
            def _():
                start(g + 1, 1 - b)

            compute(g, b)
        return carry

    lax.fori_loop(0, (_G - 1) // 2, lambda i, c: step2(i * 2, c), 0)
    wait(0)
    compute(_G - 1, 0)

    pltpu.sync_copy(out_all, out_hbm.at[pl.ds(base, _EW)])


@jax.jit
def _run(z, src, dst):
    mesh = plsc.VectorSubcoreMesh(core_axis_name="c", subcore_axis_name="s")
    f = pl.kernel(
        _sc_body,
        out_type=jax.ShapeDtypeStruct((_E,), jnp.float32),
        mesh=mesh,
        compiler_params=pltpu.CompilerParams(needs_layout_passes=False),
        scratch_types=[
            pltpu.VMEM((_EW,), jnp.int32),
            pltpu.VMEM((_EW,), jnp.int32),
            pltpu.VMEM((2, _C, _D), jnp.float32),
            pltpu.VMEM((2, _C, _D), jnp.float32),
            pltpu.VMEM((_EW,), jnp.float32),
            pltpu.VMEM((_L * _L,), jnp.float32),
            pltpu.SemaphoreType.DMA,
            pltpu.SemaphoreType.DMA,
            pltpu.SemaphoreType.DMA,
            pltpu.SemaphoreType.DMA,
        ],
    )
    return f(z, src, dst)


def kernel(z, edge_index):
    src = edge_index[0].astype(jnp.int32)
    dst = edge_index[1].astype(jnp.int32)
    return _run(z, src, dst)
